# baseline (speedup 1.0000x reference)
import functools

import jax
import jax.numpy as jnp
from jax import lax
from jax.experimental import pallas as pl
from jax.experimental.pallas import tpu as pltpu
from jax.experimental.pallas import tpu_sc as plsc

NC = 2
NS = 16
LANES = 16
NWORK = NC * NS
D = 128
KD = D // LANES
L1 = 10001
LPAD = 10240


IW = 128
NSLOT = 2


def _make_spmm(nnz_w, l_pad, interpret=False):
    n = nnz_w // IW
    rpt = l_pad // NS
    mesh = plsc.VectorSubcoreMesh(core_axis_name="c", subcore_axis_name="s",
                                  num_cores=NC, num_subcores=NS)

    @functools.partial(
        pl.kernel,
        out_type=jax.ShapeDtypeStruct((NC, l_pad, D), jnp.float32),
        mesh=mesh,
        interpret=interpret,
        compiler_params=pltpu.CompilerParams(use_tc_tiling_on_sc=False),
        scratch_types=[
            pltpu.VMEM((NSLOT, IW), jnp.int32),
            pltpu.VMEM((NSLOT, IW), jnp.int32),
            pltpu.VMEM((NSLOT * IW,), jnp.float32),
            pltpu.VMEM((NSLOT * IW, D), jnp.float32),
            pltpu.VMEM_SHARED((l_pad, D), jnp.float32),
            pltpu.SemaphoreType.DMA,
            pltpu.SemaphoreType.DMA,
        ],
    )
    def spmm(rows_hbm, cols_hbm, vals_hbm, src_hbm, zeros_hbm, out_hbm,
             rowv, colv, valv, gbuf, accum, sem_i, sem_g):
        cid = lax.axis_index("c")
        sid = lax.axis_index("s")
        w = cid * NS + sid
        blk0 = w * n

        def idx_issue(b, slot):
            vbase = pl.multiple_of(b * IW, IW)
            pltpu.async_copy(cols_hbm.at[pl.ds(b, 1)],
                             colv.at[pl.ds(slot, 1)], sem_i)
            pltpu.async_copy(rows_hbm.at[pl.ds(b, 1)],
                             rowv.at[pl.ds(slot, 1)], sem_i)
            pltpu.async_copy(vals_hbm.at[pl.ds(vbase, IW)],
                             valv.at[pl.ds(slot * IW, IW)], sem_i)

        def idx_wait(b, slot):
            vbase = pl.multiple_of(b * IW, IW)
            pltpu.make_async_copy(cols_hbm.at[pl.ds(b, 1)],
                                  colv.at[pl.ds(slot, 1)], sem_i).wait()
            pltpu.make_async_copy(rows_hbm.at[pl.ds(b, 1)],
                                  rowv.at[pl.ds(slot, 1)], sem_i).wait()
            pltpu.make_async_copy(vals_hbm.at[pl.ds(vbase, IW)],
                                  valv.at[pl.ds(slot * IW, IW)], sem_i).wait()

        def gather_issue(slot):
            pltpu.async_copy(src_hbm.at[colv.at[slot]],
                             gbuf.at[pl.ds(slot * IW, IW)], sem_g)

        def gather_wait(slot):
            pltpu.make_async_copy(src_hbm.at[colv.at[slot]],
                                  gbuf.at[pl.ds(slot * IW, IW)], sem_g).wait()

        def scale_scatter(slot):
            gb = slot * IW

            def scale(i, inner):
                row0 = gb + i * LANES
                vvec = valv[pl.ds(slot * IW + i * LANES, LANES)]
                for j in range(LANES):
                    v = vvec[j]
                    for k in range(KD):
                        sl = pl.ds(k * LANES, LANES)
                        gbuf[row0 + j, sl] = gbuf[row0 + j, sl] * v
                return inner

            lax.fori_loop(0, IW // LANES, scale, 0)
            pltpu.sync_copy(gbuf.at[pl.ds(gb, IW)],
                            accum.at[rowv.at[slot]], add=True)

        pltpu.sync_copy(zeros_hbm, accum.at[pl.ds(sid * rpt, rpt)])
        plsc.subcore_barrier()

        idx_issue(blk0, 0)
        idx_issue(blk0 + 1, 1)
        idx_wait(blk0, 0)
        gather_issue(0)

        def body(i, carry):
            s = lax.rem(i, 2)
            s1 = lax.rem(i + 1, 2)
            gather_wait(s)
            idx_wait(blk0 + i + 1, s1)
            gather_issue(s1)
            scale_scatter(s)
            idx_issue(blk0 + jnp.minimum(i + 2, n - 1), s)
            return carry

        lax.fori_loop(0, n - 1, body, 0)

        last = (n - 1) % 2
        gather_wait(last)
        scale_scatter(last)
        idx_wait(blk0 + n - 1, n % 2)

        plsc.subcore_barrier()
        pltpu.sync_copy(accum.at[pl.ds(sid * rpt, rpt)],
                        out_hbm.at[cid, pl.ds(sid * rpt, rpt)])

    return spmm


LP = 10016
RPW = LP // NWORK
IW2 = 64


def _make_spmm_resident(nnz_pad, interpret=False):
    nb = nnz_pad // IW2
    spt = LP // NS
    mesh = plsc.VectorSubcoreMesh(core_axis_name="c", subcore_axis_name="s",
                                  num_cores=NC, num_subcores=NS)

    @functools.partial(
        pl.kernel,
        out_type=jax.ShapeDtypeStruct((LP, D), jnp.float32),
        mesh=mesh,
        interpret=interpret,
        compiler_params=pltpu.CompilerParams(use_tc_tiling_on_sc=False),
        scratch_types=[
            pltpu.VMEM((NSLOT, IW2), jnp.int32),
            pltpu.VMEM((NSLOT, IW2), jnp.int32),
            pltpu.VMEM((NSLOT * IW2,), jnp.float32),
            pltpu.VMEM((16,), jnp.int32),
            pltpu.VMEM((IW2, D), jnp.float32),
            pltpu.VMEM((RPW, D), jnp.float32),
            pltpu.VMEM_SHARED((LP, D), jnp.float32),
            pltpu.SemaphoreType.DMA,
        ],
    )
    def spmm(rows_hbm, cols_hbm, vals_hbm, src_hbm, sa_hbm, sb_hbm,
             zeros_hbm, out_hbm, rowv, colv, valv, sbv, gbuf, accum,
             src_sh, sem_i):
        cid = lax.axis_index("c")
        sid = lax.axis_index("s")
        w = cid * NS + sid
        lo = w * RPW
        nbc = nb - 1

        pltpu.sync_copy(src_hbm.at[pl.ds(sid * spt, spt)],
                        src_sh.at[pl.ds(sid * spt, spt)])
        pltpu.sync_copy(zeros_hbm, accum)

        pltpu.sync_copy(sa_hbm.at[pl.ds(w * 16, 16)], sbv)
        start = sbv[0]
        pltpu.sync_copy(sb_hbm.at[pl.ds(w * 16, 16)], sbv)
        end = sbv[0]
        b_lo = lax.div(start, IW2)
        n_w = lax.div(end + IW2 - 1, IW2) - b_lo
        n_eff = jnp.maximum(n_w, 1)

        def idx_issue(b, slot):
            vbase = pl.multiple_of(b * IW2, IW2)
            pltpu.async_copy(cols_hbm.at[pl.ds(b, 1)],
                             colv.at[pl.ds(slot, 1)], sem_i)
            pltpu.async_copy(rows_hbm.at[pl.ds(b, 1)],
                             rowv.at[pl.ds(slot, 1)], sem_i)
            pltpu.async_copy(vals_hbm.at[pl.ds(vbase, IW2)],
                             valv.at[pl.ds(slot * IW2, IW2)], sem_i)

        def idx_wait(b, slot):
            vbase = pl.multiple_of(b * IW2, IW2)
            pltpu.make_async_copy(cols_hbm.at[pl.ds(b, 1)],
                                  colv.at[pl.ds(slot, 1)], sem_i).wait()
            pltpu.make_async_copy(rows_hbm.at[pl.ds(b, 1)],
                                  rowv.at[pl.ds(slot, 1)], sem_i).wait()
            pltpu.make_async_copy(vals_hbm.at[pl.ds(vbase, IW2)],
                                  valv.at[pl.ds(slot * IW2, IW2)],
                                  sem_i).wait()

        plsc.subcore_barrier()

        idx_issue(jnp.minimum(b_lo, nbc), 0)
        idx_issue(jnp.minimum(b_lo + 1, nbc), 1)

        def body(i, carry):
            s = lax.rem(i, 2)
            idx_wait(jnp.minimum(b_lo + i, nbc), s)
            pltpu.sync_copy(src_sh.at[colv.at[s]], gbuf)

            def scale(g, inner):
                row0 = g * LANES
                sl16 = pl.ds(s * IW2 + row0, LANES)
                rvec = rowv[s, pl.ds(row0, LANES)]
                vvec = valv[sl16]
                msk = jnp.logical_and(rvec >= lo, rvec < lo + RPW)
                vv = jnp.where(msk, vvec, 0.0)
                rowv[s, pl.ds(row0, LANES)] = jnp.clip(rvec - lo, 0, RPW - 1)
                for j in range(LANES):
                    v = vv[j]
                    for k in range(KD):
                        sl = pl.ds(k * LANES, LANES)
                        gbuf[row0 + j, sl] = gbuf[row0 + j, sl] * v
                return inner

            lax.fori_loop(0, IW2 // LANES, scale, 0)
            pltpu.sync_copy(gbuf, accum.at[rowv.at[s]], add=True)
            idx_issue(jnp.minimum(b_lo + i + 2, nbc), s)
            return carry

        lax.fori_loop(0, n_eff, body, 0)

        idx_wait(jnp.minimum(b_lo + n_eff, nbc), lax.rem(n_eff, 2))
        idx_wait(jnp.minimum(b_lo + n_eff + 1, nbc), lax.rem(n_eff + 1, 2))

        pltpu.sync_copy(accum, out_hbm.at[pl.ds(w * RPW, RPW)])

    return spmm


def _doc_encoder(e0, e1, w1, w2):
    b = e0.shape[0]
    blk = 128

    def body(e0_ref, e1_ref, w1_ref, w2_ref, o_ref):
        e = e0_ref[...] + e1_ref[...]
        h = jnp.maximum(
            jnp.dot(e, w1_ref[...], preferred_element_type=jnp.float32), 0.0)
        o_ref[...] = jnp.dot(
            h, w2_ref[...], preferred_element_type=jnp.float32) + e

    return pl.pallas_call(
        body,
        grid=(b // blk,),
        in_specs=[
            pl.BlockSpec((blk, D), lambda i: (i, 0)),
            pl.BlockSpec((blk, D), lambda i: (i, 0)),
            pl.BlockSpec((D, D), lambda i: (0, 0)),
            pl.BlockSpec((D, D), lambda i: (0, 0)),
        ],
        out_specs=pl.BlockSpec((blk, D), lambda i: (i, 0)),
        out_shape=jax.ShapeDtypeStruct((b, D), jnp.float32),
    )(e0, e1, w1, w2)


def _combine(p0, p1, normalize):
    n = p0.shape[0]
    blk = 1024

    def body(a_ref, b_ref, o_ref):
        s = a_ref[...] + b_ref[...]
        if normalize:
            nrm = jnp.sqrt(jnp.sum(s * s, axis=1, keepdims=True))
            s = s / (nrm + 1e-8)
        o_ref[...] = s

    return pl.pallas_call(
        body,
        grid=(n // blk,),
        in_specs=[
            pl.BlockSpec((blk, D), lambda i: (i, 0)),
            pl.BlockSpec((blk, D), lambda i: (i, 0)),
        ],
        out_specs=pl.BlockSpec((blk, D), lambda i: (i, 0)),
        out_shape=jax.ShapeDtypeStruct((n, D), jnp.float32),
    )(p0, p1)


def _final(alpha2d, doc, l0, h1, p0, p1, wg1, wg2):
    b = doc.shape[0]
    n = l0.shape[0]
    blk = 512

    def body(alpha_ref, doc_ref, l0_ref, h1_ref, p0_ref, p1_ref,
             wg1_ref, wg2_ref, o_ref):
        a0 = alpha_ref[0, 0]
        a1 = alpha_ref[0, 1]
        a2 = alpha_ref[0, 2]
        lbl = (a0 * l0_ref[...] + a1 * h1_ref[...]
               + a2 * (p0_ref[...] + p1_ref[...]))
        z = jnp.maximum(
            jnp.dot(lbl, wg1_ref[...], preferred_element_type=jnp.float32),
            0.0)
        z = jnp.dot(z, wg2_ref[...], preferred_element_type=jnp.float32) + lbl
        o_ref[...] = lax.dot_general(
            doc_ref[...], z, (((1,), (1,)), ((), ())),
            preferred_element_type=jnp.float32)

    return pl.pallas_call(
        body,
        grid=(n // blk,),
        in_specs=[
            pl.BlockSpec(memory_space=pltpu.SMEM),
            pl.BlockSpec((b, D), lambda j: (0, 0)),
            pl.BlockSpec((blk, D), lambda j: (j, 0)),
            pl.BlockSpec((blk, D), lambda j: (j, 0)),
            pl.BlockSpec((blk, D), lambda j: (j, 0)),
            pl.BlockSpec((blk, D), lambda j: (j, 0)),
            pl.BlockSpec((D, D), lambda j: (0, 0)),
            pl.BlockSpec((D, D), lambda j: (0, 0)),
        ],
        out_specs=pl.BlockSpec((b, blk), lambda j: (0, j)),
        out_shape=jax.ShapeDtypeStruct((b, n), jnp.float32),
    )(alpha2d, doc, l0, h1, p0, p1, wg1, wg2)


def _pad_coo(rows, cols, vals):
    nnz = rows.shape[0]
    nnz_w = max(2, -(-nnz // (NWORK * IW))) * IW
    pad = NWORK * nnz_w - nnz
    rows = jnp.pad(rows, (0, pad)).reshape(-1, IW).astype(jnp.int32)
    cols = jnp.pad(cols, (0, pad)).reshape(-1, IW).astype(jnp.int32)
    vals = jnp.pad(vals, (0, pad))
    return rows, cols, vals, nnz_w


def kernel(X, X_w, lw_rows, lw_cols, lw_vals, g_rows, g_cols, g_vals,
           table, W1, W2, Wg1, Wg2, alpha):
    b, ldoc = X.shape
    mpad = LPAD + b

    doc_rows = jnp.repeat(jnp.arange(b, dtype=jnp.int32), ldoc) + LPAD
    m_rows, m_cols, m_vals, nnz_w_m = _pad_coo(
        jnp.concatenate([lw_rows.astype(jnp.int32), doc_rows]),
        jnp.concatenate([lw_cols.astype(jnp.int32),
                         X.reshape(-1).astype(jnp.int32)]),
        jnp.concatenate([lw_vals, X_w.reshape(-1)]))
    gg_rows, gg_cols, gg_vals, nnz_w_g = _pad_coo(g_rows, g_cols, g_vals)

    zeros_m = jnp.zeros((mpad // NS, D), jnp.float32)
    zeros_lbl = jnp.zeros((LPAD // NS, D), jnp.float32)

    spmm_m = _make_spmm(nnz_w_m, mpad)
    spmm_g = _make_spmm(nnz_w_g, LPAD)

    m_p = spmm_m(m_rows, m_cols, m_vals, table, zeros_m)

    doc = _doc_encoder(m_p[0, LPAD:], m_p[1, LPAD:], W1, W2)
    lbl0n = _combine(m_p[0, :LPAD], m_p[1, :LPAD], normalize=True)

    h1_p = spmm_g(gg_rows, gg_cols, gg_vals, lbl0n, zeros_lbl)
    hop1 = _combine(h1_p[0], h1_p[1], normalize=False)
    h2_p = spmm_g(gg_rows, gg_cols, gg_vals, hop1, zeros_lbl)

    out = _final(alpha.reshape(1, 3), doc, lbl0n, hop1, h2_p[0], h2_p[1],
                 Wg1, Wg2)
    return out[:, :L1]


# speedup vs baseline: 1.3139x; 1.3139x over previous
import functools

import jax
import jax.numpy as jnp
from jax import lax
from jax.experimental import pallas as pl
from jax.experimental.pallas import tpu as pltpu
from jax.experimental.pallas import tpu_sc as plsc

NC = 2
NS = 16
LANES = 16
NWORK = NC * NS
D = 128
KD = D // LANES
L1 = 10001
LPAD = 10240


IW = 128
NSLOT = 2


def _make_spmm(nnz_w, l_pad, interpret=False):
    n = nnz_w // IW
    rpt = l_pad // NS
    mesh = plsc.VectorSubcoreMesh(core_axis_name="c", subcore_axis_name="s",
                                  num_cores=NC, num_subcores=NS)

    @functools.partial(
        pl.kernel,
        out_type=jax.ShapeDtypeStruct((NC, l_pad, D), jnp.float32),
        mesh=mesh,
        interpret=interpret,
        compiler_params=pltpu.CompilerParams(use_tc_tiling_on_sc=False),
        scratch_types=[
            pltpu.VMEM((NSLOT, IW), jnp.int32),
            pltpu.VMEM((NSLOT, IW), jnp.int32),
            pltpu.VMEM((NSLOT * IW,), jnp.float32),
            pltpu.VMEM((NSLOT * IW, D), jnp.float32),
            pltpu.VMEM_SHARED((l_pad, D), jnp.float32),
            pltpu.SemaphoreType.DMA,
            pltpu.SemaphoreType.DMA,
        ],
    )
    def spmm(rows_hbm, cols_hbm, vals_hbm, src_hbm, zeros_hbm, out_hbm,
             rowv, colv, valv, gbuf, accum, sem_i, sem_g):
        cid = lax.axis_index("c")
        sid = lax.axis_index("s")
        w = cid * NS + sid
        blk0 = w * n

        def idx_issue(b, slot):
            vbase = pl.multiple_of(b * IW, IW)
            pltpu.async_copy(cols_hbm.at[pl.ds(b, 1)],
                             colv.at[pl.ds(slot, 1)], sem_i)
            pltpu.async_copy(rows_hbm.at[pl.ds(b, 1)],
                             rowv.at[pl.ds(slot, 1)], sem_i)
            pltpu.async_copy(vals_hbm.at[pl.ds(vbase, IW)],
                             valv.at[pl.ds(slot * IW, IW)], sem_i)

        def idx_wait(b, slot):
            vbase = pl.multiple_of(b * IW, IW)
            pltpu.make_async_copy(cols_hbm.at[pl.ds(b, 1)],
                                  colv.at[pl.ds(slot, 1)], sem_i).wait()
            pltpu.make_async_copy(rows_hbm.at[pl.ds(b, 1)],
                                  rowv.at[pl.ds(slot, 1)], sem_i).wait()
            pltpu.make_async_copy(vals_hbm.at[pl.ds(vbase, IW)],
                                  valv.at[pl.ds(slot * IW, IW)], sem_i).wait()

        def gather_issue(slot):
            pltpu.async_copy(src_hbm.at[colv.at[slot]],
                             gbuf.at[pl.ds(slot * IW, IW)], sem_g)

        def gather_wait(slot):
            pltpu.make_async_copy(src_hbm.at[colv.at[slot]],
                                  gbuf.at[pl.ds(slot * IW, IW)], sem_g).wait()

        def scale_scatter(slot):
            gb = slot * IW

            def scale(i, inner):
                row0 = gb + i * LANES
                vvec = valv[pl.ds(slot * IW + i * LANES, LANES)]
                for j in range(LANES):
                    v = vvec[j]
                    for k in range(KD):
                        sl = pl.ds(k * LANES, LANES)
                        gbuf[row0 + j, sl] = gbuf[row0 + j, sl] * v
                return inner

            lax.fori_loop(0, IW // LANES, scale, 0)
            pltpu.sync_copy(gbuf.at[pl.ds(gb, IW)],
                            accum.at[rowv.at[slot]], add=True)

        pltpu.sync_copy(zeros_hbm, accum.at[pl.ds(sid * rpt, rpt)])
        plsc.subcore_barrier()

        idx_issue(blk0, 0)
        idx_issue(blk0 + 1, 1)
        idx_wait(blk0, 0)
        gather_issue(0)

        def body(i, carry):
            s = lax.rem(i, 2)
            s1 = lax.rem(i + 1, 2)
            gather_wait(s)
            idx_wait(blk0 + i + 1, s1)
            gather_issue(s1)
            scale_scatter(s)
            idx_issue(blk0 + jnp.minimum(i + 2, n - 1), s)
            return carry

        lax.fori_loop(0, n - 1, body, 0)

        last = (n - 1) % 2
        gather_wait(last)
        scale_scatter(last)
        idx_wait(blk0 + n - 1, n % 2)

        plsc.subcore_barrier()
        pltpu.sync_copy(accum.at[pl.ds(sid * rpt, rpt)],
                        out_hbm.at[cid, pl.ds(sid * rpt, rpt)])

    return spmm


LP = 10016
RPW = LP // NWORK
IW2 = 64


def _make_spmm_resident(nnz_pad, interpret=False):
    nb = nnz_pad // IW2
    crows = NS * RPW
    mesh = plsc.VectorSubcoreMesh(core_axis_name="c", subcore_axis_name="s",
                                  num_cores=NC, num_subcores=NS)

    @functools.partial(
        pl.kernel,
        out_type=jax.ShapeDtypeStruct((LP, D), jnp.float32),
        mesh=mesh,
        interpret=interpret,
        compiler_params=pltpu.CompilerParams(use_tc_tiling_on_sc=False),
        scratch_types=[
            pltpu.VMEM((NSLOT, IW2), jnp.int32),
            pltpu.VMEM((NSLOT, IW2), jnp.int32),
            pltpu.VMEM((NSLOT * IW2,), jnp.float32),
            pltpu.VMEM((16,), jnp.int32),
            pltpu.VMEM((NSLOT * IW2, D), jnp.float32),
            pltpu.VMEM_SHARED((NS * RPW, D), jnp.float32),
            pltpu.SemaphoreType.DMA,
            pltpu.SemaphoreType.DMA,
        ],
    )
    def spmm(rows_hbm, cols_hbm, vals_hbm, src_hbm, sa_hbm, sb_hbm,
             zeros_hbm, out_hbm, rowv, colv, valv, sbv, gbuf, accum,
             sem_i, sem_g):
        cid = lax.axis_index("c")
        sid = lax.axis_index("s")
        w = cid * NS + sid
        lo = w * RPW
        clo = cid * crows
        nbc = nb - 1

        pltpu.sync_copy(zeros_hbm, accum.at[pl.ds(sid * RPW, RPW)])

        pltpu.sync_copy(sa_hbm.at[pl.ds(w * 16, 16)], sbv)
        start = sbv[pl.ds(0, 16)][0]
        pltpu.sync_copy(sb_hbm.at[pl.ds(w * 16, 16)], sbv)
        end = sbv[pl.ds(0, 16)][0]
        b_lo = lax.div(start, IW2)
        n_w = lax.div(end + IW2 - 1, IW2) - b_lo
        n_eff = jnp.maximum(n_w, 1)

        def blki(i):
            return jnp.minimum(b_lo + i, nbc)

        def idx_issue(b, slot):
            vbase = b * IW2
            pltpu.async_copy(cols_hbm.at[pl.ds(b, 1)],
                             colv.at[pl.ds(slot, 1)], sem_i)
            pltpu.async_copy(rows_hbm.at[pl.ds(b, 1)],
                             rowv.at[pl.ds(slot, 1)], sem_i)
            pltpu.async_copy(vals_hbm.at[pl.ds(vbase, IW2)],
                             valv.at[pl.ds(slot * IW2, IW2)], sem_i)

        def idx_wait(b, slot):
            vbase = b * IW2
            pltpu.make_async_copy(cols_hbm.at[pl.ds(b, 1)],
                                  colv.at[pl.ds(slot, 1)], sem_i).wait()
            pltpu.make_async_copy(rows_hbm.at[pl.ds(b, 1)],
                                  rowv.at[pl.ds(slot, 1)], sem_i).wait()
            pltpu.make_async_copy(vals_hbm.at[pl.ds(vbase, IW2)],
                                  valv.at[pl.ds(slot * IW2, IW2)],
                                  sem_i).wait()

        def gather_issue(slot):
            pltpu.async_copy(src_hbm.at[colv.at[slot]],
                             gbuf.at[pl.ds(slot * IW2, IW2)], sem_g)

        def gather_wait(slot):
            pltpu.make_async_copy(src_hbm.at[colv.at[slot]],
                                  gbuf.at[pl.ds(slot * IW2, IW2)],
                                  sem_g).wait()

        def scale_scatter(slot):
            gb = slot * IW2

            def scale(g, inner):
                row0 = g * LANES
                rvec = rowv[slot, pl.ds(row0, LANES)]
                vvec = valv[pl.ds(slot * IW2 + row0, LANES)]
                msk = jnp.logical_and(rvec >= lo, rvec < lo + RPW)
                vv = jnp.where(msk, vvec, 0.0)
                rowv[slot, pl.ds(row0, LANES)] = jnp.clip(
                    rvec - clo, 0, crows - 1)
                for j in range(LANES):
                    v = vv[j]
                    for k in range(KD):
                        sl = pl.ds(k * LANES, LANES)
                        gbuf[gb + row0 + j, sl] = gbuf[gb + row0 + j, sl] * v
                return inner

            lax.fori_loop(0, IW2 // LANES, scale, 0)
            pltpu.sync_copy(gbuf.at[pl.ds(gb, IW2)],
                            accum.at[rowv.at[slot]], add=True)

        plsc.subcore_barrier()

        idx_issue(blki(0), 0)
        idx_issue(blki(1), 1)
        idx_wait(blki(0), 0)
        gather_issue(0)

        def body(i, carry):
            s = lax.rem(i, 2)
            s1 = lax.rem(i + 1, 2)
            gather_wait(s)
            idx_wait(blki(i + 1), s1)
            gather_issue(s1)
            scale_scatter(s)
            idx_issue(blki(i + 2), s)
            return carry

        lax.fori_loop(0, n_eff - 1, body, 0)

        last = lax.rem(n_eff - 1, 2)
        gather_wait(last)
        scale_scatter(last)
        idx_wait(blki(n_eff), lax.rem(n_eff, 2))

        plsc.subcore_barrier()
        pltpu.sync_copy(accum.at[pl.ds(sid * RPW, RPW)],
                        out_hbm.at[pl.ds(w * RPW, RPW)])

    return spmm


def _doc_encoder(e0, e1, w1, w2):
    b = e0.shape[0]
    blk = 128

    def body(e0_ref, e1_ref, w1_ref, w2_ref, o_ref):
        e = e0_ref[...] + e1_ref[...]
        h = jnp.maximum(
            jnp.dot(e, w1_ref[...], preferred_element_type=jnp.float32), 0.0)
        o_ref[...] = jnp.dot(
            h, w2_ref[...], preferred_element_type=jnp.float32) + e

    return pl.pallas_call(
        body,
        grid=(b // blk,),
        in_specs=[
            pl.BlockSpec((blk, D), lambda i: (i, 0)),
            pl.BlockSpec((blk, D), lambda i: (i, 0)),
            pl.BlockSpec((D, D), lambda i: (0, 0)),
            pl.BlockSpec((D, D), lambda i: (0, 0)),
        ],
        out_specs=pl.BlockSpec((blk, D), lambda i: (i, 0)),
        out_shape=jax.ShapeDtypeStruct((b, D), jnp.float32),
    )(e0, e1, w1, w2)


def _combine(p0, p1, normalize):
    n = p0.shape[0]
    blk = 1024

    def body(a_ref, b_ref, o_ref):
        s = a_ref[...] + b_ref[...]
        if normalize:
            nrm = jnp.sqrt(jnp.sum(s * s, axis=1, keepdims=True))
            s = s / (nrm + 1e-8)
        o_ref[...] = s

    return pl.pallas_call(
        body,
        grid=(n // blk,),
        in_specs=[
            pl.BlockSpec((blk, D), lambda i: (i, 0)),
            pl.BlockSpec((blk, D), lambda i: (i, 0)),
        ],
        out_specs=pl.BlockSpec((blk, D), lambda i: (i, 0)),
        out_shape=jax.ShapeDtypeStruct((n, D), jnp.float32),
    )(p0, p1)


def _final(alpha2d, doc, l0, h1, h2, wg1, wg2):
    b = doc.shape[0]
    n = l0.shape[0]
    blk = 512

    def body(alpha_ref, doc_ref, l0_ref, h1_ref, h2_ref,
             wg1_ref, wg2_ref, o_ref):
        a0 = alpha_ref[0, 0]
        a1 = alpha_ref[0, 1]
        a2 = alpha_ref[0, 2]
        lbl = a0 * l0_ref[...] + a1 * h1_ref[...] + a2 * h2_ref[...]
        z = jnp.maximum(
            jnp.dot(lbl, wg1_ref[...], preferred_element_type=jnp.float32),
            0.0)
        z = jnp.dot(z, wg2_ref[...], preferred_element_type=jnp.float32) + lbl
        o_ref[...] = lax.dot_general(
            doc_ref[...], z, (((1,), (1,)), ((), ())),
            preferred_element_type=jnp.float32)

    return pl.pallas_call(
        body,
        grid=(n // blk,),
        in_specs=[
            pl.BlockSpec(memory_space=pltpu.SMEM),
            pl.BlockSpec((b, D), lambda j: (0, 0)),
            pl.BlockSpec((blk, D), lambda j: (j, 0)),
            pl.BlockSpec((blk, D), lambda j: (j, 0)),
            pl.BlockSpec((blk, D), lambda j: (j, 0)),
            pl.BlockSpec((D, D), lambda j: (0, 0)),
            pl.BlockSpec((D, D), lambda j: (0, 0)),
        ],
        out_specs=pl.BlockSpec((b, blk), lambda j: (0, j)),
        out_shape=jax.ShapeDtypeStruct((b, n), jnp.float32),
    )(alpha2d, doc, l0, h1, h2, wg1, wg2)


def _pad_coo(rows, cols, vals):
    nnz = rows.shape[0]
    nnz_w = max(2, -(-nnz // (NWORK * IW))) * IW
    pad = NWORK * nnz_w - nnz
    rows = jnp.pad(rows, (0, pad)).reshape(-1, IW).astype(jnp.int32)
    cols = jnp.pad(cols, (0, pad)).reshape(-1, IW).astype(jnp.int32)
    vals = jnp.pad(vals, (0, pad))
    return rows, cols, vals, nnz_w


def _pad_coo_res(rows, cols, vals):
    nnz = rows.shape[0]
    nnz_pad = -(-nnz // IW2) * IW2
    pad = nnz_pad - nnz
    rows = jnp.pad(rows.astype(jnp.int32), (0, pad),
                   constant_values=LP - 1)
    cols = jnp.pad(cols.astype(jnp.int32), (0, pad))
    vals = jnp.pad(vals, (0, pad))
    bounds = jnp.arange(NWORK + 1, dtype=jnp.int32) * RPW
    ss = jnp.searchsorted(rows, bounds).astype(jnp.int32)
    sa = jnp.repeat(ss[:-1], 16)
    sb = jnp.repeat(ss[1:], 16)
    return (rows.reshape(-1, IW2), cols.reshape(-1, IW2), vals,
            sa, sb, nnz_pad)


def kernel(X, X_w, lw_rows, lw_cols, lw_vals, g_rows, g_cols, g_vals,
           table, W1, W2, Wg1, Wg2, alpha):
    b, ldoc = X.shape
    mpad = LPAD + b

    doc_rows = jnp.repeat(jnp.arange(b, dtype=jnp.int32), ldoc) + LPAD
    m_rows, m_cols, m_vals, nnz_w_m = _pad_coo(
        jnp.concatenate([lw_rows.astype(jnp.int32), doc_rows]),
        jnp.concatenate([lw_cols.astype(jnp.int32),
                         X.reshape(-1).astype(jnp.int32)]),
        jnp.concatenate([lw_vals, X_w.reshape(-1)]))
    gr2, gc2, gv2, sa, sb, nnz_pad_g = _pad_coo_res(g_rows, g_cols, g_vals)

    zeros_m = jnp.zeros((mpad // NS, D), jnp.float32)
    zeros_rpw = jnp.zeros((RPW, D), jnp.float32)

    spmm_m = _make_spmm(nnz_w_m, mpad)
    spmm_r = _make_spmm_resident(nnz_pad_g)

    m_p = spmm_m(m_rows, m_cols, m_vals, table, zeros_m)

    doc = _doc_encoder(m_p[0, LPAD:], m_p[1, LPAD:], W1, W2)
    lbl0n = _combine(m_p[0, :LPAD], m_p[1, :LPAD], normalize=True)

    hop1 = spmm_r(gr2, gc2, gv2, lbl0n[:LP], sa, sb, zeros_rpw)
    hop2 = spmm_r(gr2, gc2, gv2, hop1, sa, sb, zeros_rpw)
    h1p = jnp.pad(hop1, ((0, LPAD - LP), (0, 0)))
    h2p = jnp.pad(hop2, ((0, LPAD - LP), (0, 0)))

    out = _final(alpha.reshape(1, 3), doc, lbl0n, h1p, h2p, Wg1, Wg2)
    return out[:, :L1]


# speedup vs baseline: 1.4727x; 1.1209x over previous
import functools

import jax
import jax.numpy as jnp
from jax import lax
from jax.experimental import pallas as pl
from jax.experimental.pallas import tpu as pltpu
from jax.experimental.pallas import tpu_sc as plsc

NC = 2
NS = 16
LANES = 16
NWORK = NC * NS
D = 128
KD = D // LANES
L1 = 10001
LPAD = 10240


IW = 128
NSLOT = 2


def _make_spmm(nnz_w, l_pad, interpret=False):
    n = nnz_w // IW
    rpt = l_pad // NS
    mesh = plsc.VectorSubcoreMesh(core_axis_name="c", subcore_axis_name="s",
                                  num_cores=NC, num_subcores=NS)

    @functools.partial(
        pl.kernel,
        out_type=jax.ShapeDtypeStruct((NC, l_pad, D), jnp.float32),
        mesh=mesh,
        interpret=interpret,
        compiler_params=pltpu.CompilerParams(use_tc_tiling_on_sc=False),
        scratch_types=[
            pltpu.VMEM((NSLOT, IW), jnp.int32),
            pltpu.VMEM((NSLOT, IW), jnp.int32),
            pltpu.VMEM((NSLOT * IW,), jnp.float32),
            pltpu.VMEM((NSLOT * IW, D), jnp.float32),
            pltpu.VMEM_SHARED((l_pad, D), jnp.float32),
            pltpu.SemaphoreType.DMA,
            pltpu.SemaphoreType.DMA,
        ],
    )
    def spmm(rows_hbm, cols_hbm, vals_hbm, src_hbm, zeros_hbm, out_hbm,
             rowv, colv, valv, gbuf, accum, sem_i, sem_g):
        cid = lax.axis_index("c")
        sid = lax.axis_index("s")
        w = cid * NS + sid
        blk0 = w * n

        def idx_issue(b, slot):
            vbase = pl.multiple_of(b * IW, IW)
            pltpu.async_copy(cols_hbm.at[pl.ds(b, 1)],
                             colv.at[pl.ds(slot, 1)], sem_i)
            pltpu.async_copy(rows_hbm.at[pl.ds(b, 1)],
                             rowv.at[pl.ds(slot, 1)], sem_i)
            pltpu.async_copy(vals_hbm.at[pl.ds(vbase, IW)],
                             valv.at[pl.ds(slot * IW, IW)], sem_i)

        def idx_wait(b, slot):
            vbase = pl.multiple_of(b * IW, IW)
            pltpu.make_async_copy(cols_hbm.at[pl.ds(b, 1)],
                                  colv.at[pl.ds(slot, 1)], sem_i).wait()
            pltpu.make_async_copy(rows_hbm.at[pl.ds(b, 1)],
                                  rowv.at[pl.ds(slot, 1)], sem_i).wait()
            pltpu.make_async_copy(vals_hbm.at[pl.ds(vbase, IW)],
                                  valv.at[pl.ds(slot * IW, IW)], sem_i).wait()

        def gather_issue(slot):
            pltpu.async_copy(src_hbm.at[colv.at[slot]],
                             gbuf.at[pl.ds(slot * IW, IW)], sem_g)

        def gather_wait(slot):
            pltpu.make_async_copy(src_hbm.at[colv.at[slot]],
                                  gbuf.at[pl.ds(slot * IW, IW)], sem_g).wait()

        def scale_scatter(slot):
            gb = slot * IW

            def scale(i, inner):
                row0 = gb + i * LANES
                vvec = valv[pl.ds(slot * IW + i * LANES, LANES)]
                for j in range(LANES):
                    v = vvec[j]
                    for k in range(KD):
                        sl = pl.ds(k * LANES, LANES)
                        gbuf[row0 + j, sl] = gbuf[row0 + j, sl] * v
                return inner

            lax.fori_loop(0, IW // LANES, scale, 0)
            pltpu.sync_copy(gbuf.at[pl.ds(gb, IW)],
                            accum.at[rowv.at[slot]], add=True)

        pltpu.sync_copy(zeros_hbm, accum.at[pl.ds(sid * rpt, rpt)])
        plsc.subcore_barrier()

        idx_issue(blk0, 0)
        idx_issue(blk0 + 1, 1)
        idx_wait(blk0, 0)
        gather_issue(0)

        def body(i, carry):
            s = lax.rem(i, 2)
            s1 = lax.rem(i + 1, 2)
            gather_wait(s)
            idx_wait(blk0 + i + 1, s1)
            gather_issue(s1)
            scale_scatter(s)
            idx_issue(blk0 + jnp.minimum(i + 2, n - 1), s)
            return carry

        lax.fori_loop(0, n - 1, body, 0)

        last = (n - 1) % 2
        gather_wait(last)
        scale_scatter(last)
        idx_wait(blk0 + n - 1, n % 2)

        plsc.subcore_barrier()
        pltpu.sync_copy(accum.at[pl.ds(sid * rpt, rpt)],
                        out_hbm.at[cid, pl.ds(sid * rpt, rpt)])

    return spmm


LP = 10016
RPW = LP // NWORK
IW2 = 128


def _make_spmm_resident(nnz_pad, interpret=False):
    nb = nnz_pad // IW2
    crows = NS * RPW
    mesh = plsc.VectorSubcoreMesh(core_axis_name="c", subcore_axis_name="s",
                                  num_cores=NC, num_subcores=NS)

    @functools.partial(
        pl.kernel,
        out_type=jax.ShapeDtypeStruct((LP, D), jnp.float32),
        mesh=mesh,
        interpret=interpret,
        compiler_params=pltpu.CompilerParams(use_tc_tiling_on_sc=False),
        scratch_types=[
            pltpu.VMEM((NSLOT, IW2), jnp.int32),
            pltpu.VMEM((NSLOT, IW2), jnp.int32),
            pltpu.VMEM((NSLOT * IW2,), jnp.float32),
            pltpu.VMEM((16,), jnp.int32),
            pltpu.VMEM((NSLOT * IW2, D), jnp.float32),
            pltpu.VMEM_SHARED((NS * RPW, D), jnp.float32),
            pltpu.SemaphoreType.DMA,
            pltpu.SemaphoreType.DMA,
        ],
    )
    def spmm(rows_hbm, cols_hbm, vals_hbm, src_hbm, sa_hbm, sb_hbm,
             zeros_hbm, out_hbm, rowv, colv, valv, sbv, gbuf, accum,
             sem_i, sem_g):
        cid = lax.axis_index("c")
        sid = lax.axis_index("s")
        w = cid * NS + sid
        lo = w * RPW
        clo = cid * crows
        nbc = nb - 1

        pltpu.sync_copy(zeros_hbm, accum.at[pl.ds(sid * RPW, RPW)])

        pltpu.sync_copy(sa_hbm.at[pl.ds(w * 16, 16)], sbv)
        start = sbv[pl.ds(0, 16)][0]
        pltpu.sync_copy(sb_hbm.at[pl.ds(w * 16, 16)], sbv)
        end = sbv[pl.ds(0, 16)][0]
        b_lo = lax.div(start, IW2)
        n_w = lax.div(end + IW2 - 1, IW2) - b_lo
        n_eff = jnp.maximum(n_w, 1)

        def blki(i):
            return jnp.minimum(b_lo + i, nbc)

        def idx_issue(b, slot):
            vbase = b * IW2
            pltpu.async_copy(cols_hbm.at[pl.ds(b, 1)],
                             colv.at[pl.ds(slot, 1)], sem_i)
            pltpu.async_copy(rows_hbm.at[pl.ds(b, 1)],
                             rowv.at[pl.ds(slot, 1)], sem_i)
            pltpu.async_copy(vals_hbm.at[pl.ds(vbase, IW2)],
                             valv.at[pl.ds(slot * IW2, IW2)], sem_i)

        def idx_wait(b, slot):
            vbase = b * IW2
            pltpu.make_async_copy(cols_hbm.at[pl.ds(b, 1)],
                                  colv.at[pl.ds(slot, 1)], sem_i).wait()
            pltpu.make_async_copy(rows_hbm.at[pl.ds(b, 1)],
                                  rowv.at[pl.ds(slot, 1)], sem_i).wait()
            pltpu.make_async_copy(vals_hbm.at[pl.ds(vbase, IW2)],
                                  valv.at[pl.ds(slot * IW2, IW2)],
                                  sem_i).wait()

        def gather_issue(slot):
            pltpu.async_copy(src_hbm.at[colv.at[slot]],
                             gbuf.at[pl.ds(slot * IW2, IW2)], sem_g)

        def gather_wait(slot):
            pltpu.make_async_copy(src_hbm.at[colv.at[slot]],
                                  gbuf.at[pl.ds(slot * IW2, IW2)],
                                  sem_g).wait()

        def scale_scatter(slot):
            gb = slot * IW2

            def scale(g, inner):
                row0 = g * LANES
                rvec = rowv[slot, pl.ds(row0, LANES)]
                vvec = valv[pl.ds(slot * IW2 + row0, LANES)]
                msk = jnp.logical_and(rvec >= lo, rvec < lo + RPW)
                vv = jnp.where(msk, vvec, 0.0)
                rowv[slot, pl.ds(row0, LANES)] = jnp.clip(
                    rvec - clo, 0, crows - 1)
                for j in range(LANES):
                    v = vv[j]
                    for k in range(KD):
                        sl = pl.ds(k * LANES, LANES)
                        gbuf[gb + row0 + j, sl] = gbuf[gb + row0 + j, sl] * v
                return inner

            lax.fori_loop(0, IW2 // LANES, scale, 0)
            pltpu.sync_copy(gbuf.at[pl.ds(gb, IW2)],
                            accum.at[rowv.at[slot]], add=True)

        plsc.subcore_barrier()

        idx_issue(blki(0), 0)
        idx_issue(blki(1), 1)
        idx_wait(blki(0), 0)
        gather_issue(0)

        def body(i, carry):
            s = lax.rem(i, 2)
            s1 = lax.rem(i + 1, 2)
            gather_wait(s)
            idx_wait(blki(i + 1), s1)
            gather_issue(s1)
            scale_scatter(s)
            idx_issue(blki(i + 2), s)
            return carry

        lax.fori_loop(0, n_eff - 1, body, 0)

        last = lax.rem(n_eff - 1, 2)
        gather_wait(last)
        scale_scatter(last)
        idx_wait(blki(n_eff), lax.rem(n_eff, 2))

        plsc.subcore_barrier()
        pltpu.sync_copy(accum.at[pl.ds(sid * RPW, RPW)],
                        out_hbm.at[pl.ds(w * RPW, RPW)])

    return spmm


def _doc_encoder(e0, e1, w1, w2):
    b = e0.shape[0]
    blk = 128

    def body(e0_ref, e1_ref, w1_ref, w2_ref, o_ref):
        e = e0_ref[...] + e1_ref[...]
        h = jnp.maximum(
            jnp.dot(e, w1_ref[...], preferred_element_type=jnp.float32), 0.0)
        o_ref[...] = jnp.dot(
            h, w2_ref[...], preferred_element_type=jnp.float32) + e

    return pl.pallas_call(
        body,
        grid=(b // blk,),
        in_specs=[
            pl.BlockSpec((blk, D), lambda i: (i, 0)),
            pl.BlockSpec((blk, D), lambda i: (i, 0)),
            pl.BlockSpec((D, D), lambda i: (0, 0)),
            pl.BlockSpec((D, D), lambda i: (0, 0)),
        ],
        out_specs=pl.BlockSpec((blk, D), lambda i: (i, 0)),
        out_shape=jax.ShapeDtypeStruct((b, D), jnp.float32),
    )(e0, e1, w1, w2)


def _combine(p0, p1, normalize):
    n = p0.shape[0]
    blk = 1024

    def body(a_ref, b_ref, o_ref):
        s = a_ref[...] + b_ref[...]
        if normalize:
            nrm = jnp.sqrt(jnp.sum(s * s, axis=1, keepdims=True))
            s = s / (nrm + 1e-8)
        o_ref[...] = s

    return pl.pallas_call(
        body,
        grid=(n // blk,),
        in_specs=[
            pl.BlockSpec((blk, D), lambda i: (i, 0)),
            pl.BlockSpec((blk, D), lambda i: (i, 0)),
        ],
        out_specs=pl.BlockSpec((blk, D), lambda i: (i, 0)),
        out_shape=jax.ShapeDtypeStruct((n, D), jnp.float32),
    )(p0, p1)


def _final(alpha2d, doc, l0, h1, h2, wg1, wg2):
    b = doc.shape[0]
    n = l0.shape[0]
    blk = 512

    def body(alpha_ref, doc_ref, l0_ref, h1_ref, h2_ref,
             wg1_ref, wg2_ref, o_ref):
        a0 = alpha_ref[0, 0]
        a1 = alpha_ref[0, 1]
        a2 = alpha_ref[0, 2]
        lbl = a0 * l0_ref[...] + a1 * h1_ref[...] + a2 * h2_ref[...]
        z = jnp.maximum(
            jnp.dot(lbl, wg1_ref[...], preferred_element_type=jnp.float32),
            0.0)
        z = jnp.dot(z, wg2_ref[...], preferred_element_type=jnp.float32) + lbl
        o_ref[...] = lax.dot_general(
            doc_ref[...], z, (((1,), (1,)), ((), ())),
            preferred_element_type=jnp.float32)

    return pl.pallas_call(
        body,
        grid=(n // blk,),
        in_specs=[
            pl.BlockSpec(memory_space=pltpu.SMEM),
            pl.BlockSpec((b, D), lambda j: (0, 0)),
            pl.BlockSpec((blk, D), lambda j: (j, 0)),
            pl.BlockSpec((blk, D), lambda j: (j, 0)),
            pl.BlockSpec((blk, D), lambda j: (j, 0)),
            pl.BlockSpec((D, D), lambda j: (0, 0)),
            pl.BlockSpec((D, D), lambda j: (0, 0)),
        ],
        out_specs=pl.BlockSpec((b, blk), lambda j: (0, j)),
        out_shape=jax.ShapeDtypeStruct((b, n), jnp.float32),
    )(alpha2d, doc, l0, h1, h2, wg1, wg2)


def _pad_coo(rows, cols, vals):
    nnz = rows.shape[0]
    nnz_w = max(2, -(-nnz // (NWORK * IW))) * IW
    pad = NWORK * nnz_w - nnz
    rows = jnp.pad(rows, (0, pad)).reshape(-1, IW).astype(jnp.int32)
    cols = jnp.pad(cols, (0, pad)).reshape(-1, IW).astype(jnp.int32)
    vals = jnp.pad(vals, (0, pad))
    return rows, cols, vals, nnz_w


def _pad_coo_res(rows, cols, vals):
    nnz = rows.shape[0]
    nnz_pad = -(-nnz // IW2) * IW2
    pad = nnz_pad - nnz
    rows = jnp.pad(rows.astype(jnp.int32), (0, pad),
                   constant_values=LP - 1)
    cols = jnp.pad(cols.astype(jnp.int32), (0, pad))
    vals = jnp.pad(vals, (0, pad))
    bounds = jnp.arange(NWORK + 1, dtype=jnp.int32) * RPW
    ss = jnp.searchsorted(rows, bounds).astype(jnp.int32)
    sa = jnp.repeat(ss[:-1], 16)
    sb = jnp.repeat(ss[1:], 16)
    return (rows.reshape(-1, IW2), cols.reshape(-1, IW2), vals,
            sa, sb, nnz_pad)


def kernel(X, X_w, lw_rows, lw_cols, lw_vals, g_rows, g_cols, g_vals,
           table, W1, W2, Wg1, Wg2, alpha):
    b, ldoc = X.shape
    mpad = LPAD + b

    doc_rows = jnp.repeat(jnp.arange(b, dtype=jnp.int32), ldoc) + LPAD
    m_rows, m_cols, m_vals, nnz_w_m = _pad_coo(
        jnp.concatenate([lw_rows.astype(jnp.int32), doc_rows]),
        jnp.concatenate([lw_cols.astype(jnp.int32),
                         X.reshape(-1).astype(jnp.int32)]),
        jnp.concatenate([lw_vals, X_w.reshape(-1)]))
    gr2, gc2, gv2, sa, sb, nnz_pad_g = _pad_coo_res(g_rows, g_cols, g_vals)

    zeros_m = jnp.zeros((mpad // NS, D), jnp.float32)
    zeros_rpw = jnp.zeros((RPW, D), jnp.float32)

    spmm_m = _make_spmm(nnz_w_m, mpad)
    spmm_r = _make_spmm_resident(nnz_pad_g)

    m_p = spmm_m(m_rows, m_cols, m_vals, table, zeros_m)

    doc = _doc_encoder(m_p[0, LPAD:], m_p[1, LPAD:], W1, W2)
    lbl0n = _combine(m_p[0, :LPAD], m_p[1, :LPAD], normalize=True)

    hop1 = spmm_r(gr2, gc2, gv2, lbl0n[:LP], sa, sb, zeros_rpw)
    hop2 = spmm_r(gr2, gc2, gv2, hop1, sa, sb, zeros_rpw)
    h1p = jnp.pad(hop1, ((0, LPAD - LP), (0, 0)))
    h2p = jnp.pad(hop2, ((0, LPAD - LP), (0, 0)))

    out = _final(alpha.reshape(1, 3), doc, lbl0n, h1p, h2p, Wg1, Wg2)
    return out[:, :L1]
